# Initial kernel scaffold; baseline (speedup 1.0000x reference)
import jax
import jax.numpy as jnp
from jax.experimental import pallas as pl


def kernel(keys, query):
    raise NotImplementedError("write your pallas kernel here")


# speedup vs baseline: 3.4992x; 3.4992x over previous
import functools

import jax
import jax.numpy as jnp
from jax import lax
from jax.experimental import pallas as pl
from jax.experimental.pallas import tpu as pltpu
from jax.experimental.pallas import tpu_sc as plsc

B, H, SQ, SK, D = 8, 16, 16, 2048, 64
K = 32
ROWS = B * H * SQ
KB = 1024

NW = 32
RPW = ROWS // NW
RB = 16


def _scores_body(q_ref, k_ref, o_ref):
    q = q_ref[0]
    k = k_ref[0]
    kn2 = jnp.sum(k * k, axis=1)
    kn = jnp.sqrt(kn2)
    knm = jnp.max(kn)
    extra = jnp.sqrt(jnp.maximum(knm * knm - kn * kn, 0.0))
    k_sq = kn2 + extra * extra
    qn2 = jnp.sum(q * q, axis=1)
    qn = jnp.maximum(jnp.sqrt(qn2), 1e-6)
    r = knm / qn
    qq = r[:, None] * q
    q_sq = jnp.sum(qq * qq, axis=1)
    dots = lax.dot_general(
        qq, k, (((1,), (1,)), ((), ())),
        preferred_element_type=jnp.float32)
    o_ref[0] = 2.0 * dots - (q_sq[:, None] + k_sq[None, :])


def _scores_tc(query, keys):
    qf = query.reshape(B * H, SQ, D)
    kf = keys.reshape(B * H, SK, D)
    return pl.pallas_call(
        _scores_body,
        grid=(B * H,),
        in_specs=[
            pl.BlockSpec((1, SQ, D), lambda i: (i, 0, 0)),
            pl.BlockSpec((1, SK, D), lambda i: (i, 0, 0)),
        ],
        out_specs=pl.BlockSpec((1, SQ, SK), lambda i: (i, 0, 0)),
        out_shape=jax.ShapeDtypeStruct((B * H, SQ, SK), jnp.float32),
    )(qf, kf)


def _sort16(k, v):
    nk, sv = lax.sort((-k, v), dimension=0, num_keys=1)
    return -nk, sv


def _leaf(a, ai, b, bi):
    ak, av = _sort16(a, ai)
    bk, bv = _sort16(b, bi)
    rbk = lax.rev(bk, (0,))
    rbv = lax.rev(bv, (0,))
    m = ak >= rbk
    hk = jnp.where(m, ak, rbk)
    hv = jnp.where(m, av, rbv)
    lk = jnp.where(m, rbk, ak)
    lv = jnp.where(m, rbv, av)
    hk, hv = _sort16(hk, hv)
    lk, lv = _sort16(lk, lv)
    return hk, hv, lk, lv


def _merge32(x, y):
    xk0, xv0, xk1, xv1 = x
    yk0, yv0, yk1, yv1 = y
    ryk0 = lax.rev(yk1, (0,))
    ryv0 = lax.rev(yv1, (0,))
    ryk1 = lax.rev(yk0, (0,))
    ryv1 = lax.rev(yv0, (0,))
    m0 = xk0 >= ryk0
    h0k = jnp.where(m0, xk0, ryk0)
    h0v = jnp.where(m0, xv0, ryv0)
    m1 = xk1 >= ryk1
    h1k = jnp.where(m1, xk1, ryk1)
    h1v = jnp.where(m1, xv1, ryv1)
    m = h0k >= h1k
    uk = jnp.where(m, h0k, h1k)
    uv = jnp.where(m, h0v, h1v)
    lk = jnp.where(m, h1k, h0k)
    lv = jnp.where(m, h1v, h0v)
    uk, uv = _sort16(uk, uv)
    lk, lv = _sort16(lk, lv)
    return uk, uv, lk, lv


def _topk_sc(scores_hbm, out_hbm, buf, obuf):
    wid = lax.axis_index("s") * 2 + lax.axis_index("c")
    base = wid * RPW
    iota = lax.iota(jnp.int32, 16)

    def row_leaf(i, g):
        a = buf[i, pl.ds(32 * g, 16)]
        b = buf[i, pl.ds(32 * g + 16, 16)]
        return _leaf(a, iota + 32 * g, b, iota + 32 * g + 16)

    def row_body(i, _):
        acc0 = row_leaf(i, 0)
        acc1 = row_leaf(i, 1)

        def grp_body(g, carry):
            a0, a1 = carry
            a0 = _merge32(a0, row_leaf(i, 2 * g))
            a1 = _merge32(a1, row_leaf(i, 2 * g + 1))
            return a0, a1

        acc0, acc1 = lax.fori_loop(1, 32, grp_body, (acc0, acc1))
        uk, uv, lk, lv = _merge32(acc0, acc1)
        obuf[i, pl.ds(0, 16)] = uv
        obuf[i, pl.ds(16, 16)] = lv
        return 0

    def block_body(bi, _):
        row0 = base + bi * RB
        pltpu.sync_copy(scores_hbm.at[pl.ds(row0, RB)], buf)
        lax.fori_loop(0, RB, row_body, 0)
        pltpu.sync_copy(obuf, out_hbm.at[pl.ds(row0, RB)])
        return 0

    lax.fori_loop(0, RPW // RB, block_body, 0)


def kernel(keys, query):
    scores = _scores_tc(query, keys)
    scores = scores.reshape(ROWS, SK)
    mesh = plsc.VectorSubcoreMesh(core_axis_name="c", subcore_axis_name="s")
    topk = functools.partial(
        pl.kernel,
        out_type=jax.ShapeDtypeStruct((ROWS, K), jnp.int32),
        mesh=mesh,
        scratch_types=[
            pltpu.VMEM((RB, SK), jnp.float32),
            pltpu.VMEM((RB, K), jnp.int32),
        ],
        compiler_params=pltpu.CompilerParams(needs_layout_passes=False),
    )(_topk_sc)
    idx = topk(scores)
    return idx.reshape(B, H, SQ, K)
